# baseline (speedup 1.0000x reference)
import jax
import jax.numpy as jnp
from jax.experimental import pallas as pl
from jax.experimental.pallas import tpu as pltpu

N_E = 1024
E_DIM = 64
BETA = 0.25
B_TOTAL = 36864
BLK = 512
GRID = B_TOTAL // BLK


def _vq_body(z_ref, c_ref, zq_ref, idx_ref, loss_ref, perp_ref,
             counts_s, acc_s):
    i = pl.program_id(0)

    @pl.when(i == 0)
    def _init():
        counts_s[...] = jnp.zeros_like(counts_s)
        acc_s[...] = jnp.zeros_like(acc_s)

    z = z_ref[...]
    c = c_ref[...]

    znorm = jnp.sum(z * z, axis=1, keepdims=True)
    csq = c * c
    ones = jnp.ones((1, E_DIM), jnp.float32)
    cnorm = jax.lax.dot_general(ones, csq, (((1,), (1,)), ((), ())),
                                preferred_element_type=jnp.float32)
    scores = jax.lax.dot_general(z, c, (((1,), (1,)), ((), ())),
                                 preferred_element_type=jnp.float32)
    d = (znorm + cnorm) - 2.0 * scores

    dmin = jnp.min(d, axis=1, keepdims=True)
    iota = jax.lax.broadcasted_iota(jnp.int32, (BLK, N_E), 1)
    idxcol = jnp.min(jnp.where(d == dmin, iota, N_E), axis=1,
                     keepdims=True)

    one_hot = (iota == idxcol).astype(jnp.float32)
    z_q = jax.lax.dot_general(one_hot, c, (((1,), (0,)), ((), ())),
                              preferred_element_type=jnp.float32)

    zq_ref[...] = z + (z_q - z)
    idx_ref[...] = idxcol

    diff = z_q - z
    acc_s[...] += jnp.sum(diff * diff, axis=(0, 1), keepdims=True)
    counts_s[...] += jnp.sum(one_hot, axis=0, keepdims=True)

    @pl.when(i == GRID - 1)
    def _finalize():
        loss_ref[...] = acc_s[...] * ((1.0 + BETA) / (B_TOTAL * E_DIM))
        e_mean = counts_s[...] * (1.0 / B_TOTAL)
        ent = jnp.sum(e_mean * jnp.log(e_mean + 1e-10),
                      axis=(0, 1), keepdims=True)
        perp_ref[...] = jnp.exp(-ent)


def kernel(z, codebook, interpret=False):
    z2 = z.reshape(B_TOTAL, E_DIM)
    zq, idx, loss, perp = pl.pallas_call(
        _vq_body,
        grid=(GRID,),
        in_specs=[
            pl.BlockSpec((BLK, E_DIM), lambda i: (i, 0)),
            pl.BlockSpec((N_E, E_DIM), lambda i: (0, 0)),
        ],
        out_specs=[
            pl.BlockSpec((BLK, E_DIM), lambda i: (i, 0)),
            pl.BlockSpec((BLK, 1), lambda i: (i, 0)),
            pl.BlockSpec((1, 1), lambda i: (0, 0)),
            pl.BlockSpec((1, 1), lambda i: (0, 0)),
        ],
        out_shape=[
            jax.ShapeDtypeStruct((B_TOTAL, E_DIM), jnp.float32),
            jax.ShapeDtypeStruct((B_TOTAL, 1), jnp.int32),
            jax.ShapeDtypeStruct((1, 1), jnp.float32),
            jax.ShapeDtypeStruct((1, 1), jnp.float32),
        ],
        scratch_shapes=[
            pltpu.VMEM((1, N_E), jnp.float32),
            pltpu.VMEM((1, 1), jnp.float32),
        ],
        interpret=interpret,
    )(z2, codebook)
    return (zq, loss.reshape(()), idx, perp.reshape(()))


# speedup vs baseline: 1.0791x; 1.0791x over previous
import jax
import jax.numpy as jnp
from jax.experimental import pallas as pl
from jax.experimental.pallas import tpu as pltpu

N_E = 1024
E_DIM = 64
BETA = 0.25
B_TOTAL = 36864
BLK = 512
GRID = B_TOTAL // BLK


def _vq_body(z_ref, c_ref, zq_ref, idx_ref, loss_ref, perp_ref,
             counts_s, acc_s):
    i = pl.program_id(0)

    @pl.when(i == 0)
    def _init():
        counts_s[...] = jnp.zeros_like(counts_s)
        acc_s[...] = jnp.zeros_like(acc_s)

    z = z_ref[...]
    c = c_ref[...]

    znorm = jnp.sum(z * z, axis=1, keepdims=True)
    csq = c * c
    ones = jnp.ones((1, E_DIM), jnp.float32)
    cnorm = jax.lax.dot_general(ones, csq, (((1,), (1,)), ((), ())),
                                preferred_element_type=jnp.float32)
    sneg2 = jax.lax.dot_general(z, -2.0 * c, (((1,), (1,)), ((), ())),
                                preferred_element_type=jnp.float32)
    d = (znorm + cnorm) + sneg2

    dmin = jnp.min(d, axis=1, keepdims=True)
    iota_f = jax.lax.broadcasted_iota(jnp.int32, (BLK, N_E), 1).astype(
        jnp.float32)
    idxcol_f = jnp.min(jnp.where(d == dmin, iota_f, float(N_E)), axis=1,
                       keepdims=True)

    one_hot = (iota_f == idxcol_f).astype(jnp.bfloat16)
    z_q = jax.lax.dot_general(one_hot, c.astype(jnp.bfloat16),
                              (((1,), (0,)), ((), ())),
                              preferred_element_type=jnp.float32)

    zq_ref[...] = z + (z_q - z)
    idx_ref[...] = idxcol_f.astype(jnp.int32)

    acc_s[...] += jnp.sum(dmin, axis=(0, 1), keepdims=True)
    counts_s[...] += jnp.sum(one_hot, axis=0, keepdims=True,
                             dtype=jnp.float32)

    @pl.when(i == GRID - 1)
    def _finalize():
        loss_ref[...] = acc_s[...] * ((1.0 + BETA) / (B_TOTAL * E_DIM))
        e_mean = counts_s[...] * (1.0 / B_TOTAL)
        ent = jnp.sum(e_mean * jnp.log(e_mean + 1e-10),
                      axis=(0, 1), keepdims=True)
        perp_ref[...] = jnp.exp(-ent)


def kernel(z, codebook, interpret=False):
    z2 = z.reshape(B_TOTAL, E_DIM)
    zq, idx, loss, perp = pl.pallas_call(
        _vq_body,
        grid=(GRID,),
        in_specs=[
            pl.BlockSpec((BLK, E_DIM), lambda i: (i, 0)),
            pl.BlockSpec((N_E, E_DIM), lambda i: (0, 0)),
        ],
        out_specs=[
            pl.BlockSpec((BLK, E_DIM), lambda i: (i, 0)),
            pl.BlockSpec((BLK, 1), lambda i: (i, 0)),
            pl.BlockSpec((1, 1), lambda i: (0, 0)),
            pl.BlockSpec((1, 1), lambda i: (0, 0)),
        ],
        out_shape=[
            jax.ShapeDtypeStruct((B_TOTAL, E_DIM), jnp.float32),
            jax.ShapeDtypeStruct((B_TOTAL, 1), jnp.int32),
            jax.ShapeDtypeStruct((1, 1), jnp.float32),
            jax.ShapeDtypeStruct((1, 1), jnp.float32),
        ],
        scratch_shapes=[
            pltpu.VMEM((1, N_E), jnp.float32),
            pltpu.VMEM((1, 1), jnp.float32),
        ],
        interpret=interpret,
    )(z2, codebook)
    return (zq, loss.reshape(()), idx, perp.reshape(()))


# speedup vs baseline: 1.6807x; 1.5575x over previous
import jax
import jax.numpy as jnp
from jax.experimental import pallas as pl
from jax.experimental.pallas import tpu as pltpu

N_E = 1024
E_DIM = 64
BETA = 0.25
B_TOTAL = 36864
BLK = 4096
GRID = B_TOTAL // BLK


def _vq_body(zt_ref, z_ref, c_ref, zq_ref, idx_ref, loss_ref, perp_ref,
             counts_s, acc_s):
    i = pl.program_id(0)

    @pl.when(i == 0)
    def _init():
        counts_s[...] = jnp.zeros_like(counts_s)
        acc_s[...] = jnp.zeros_like(acc_s)

    zt = zt_ref[...]
    z = z_ref[...]
    c = c_ref[...]

    znorm_t = jnp.sum(zt * zt, axis=0, keepdims=True)
    cnorm_c = jnp.sum(c * c, axis=1, keepdims=True)

    sneg2_t = jax.lax.dot_general(-2.0 * c, z, (((1,), (1,)), ((), ())),
                                  preferred_element_type=jnp.float32)
    d_t = (znorm_t + cnorm_c) + sneg2_t

    dmin_t = jnp.min(d_t, axis=0, keepdims=True)
    iota_c = jax.lax.broadcasted_iota(jnp.int32, (N_E, 1), 0).astype(
        jnp.float32)
    idxrow_f = jnp.min(jnp.where(d_t == dmin_t, iota_c, float(N_E)),
                       axis=0, keepdims=True)

    one_hot_t = (iota_c == idxrow_f).astype(jnp.bfloat16)
    z_q = jax.lax.dot_general(one_hot_t, c.astype(jnp.bfloat16),
                              (((0,), (0,)), ((), ())),
                              preferred_element_type=jnp.float32)

    zq_ref[...] = z + (z_q - z)
    idx_ref[...] = idxrow_f.astype(jnp.int32).reshape(1, 1, BLK)

    diff = z_q - z
    acc_s[...] += jnp.sum(diff * diff, axis=(0, 1), keepdims=True)
    counts_s[...] += jnp.sum(one_hot_t, axis=1, keepdims=True,
                             dtype=jnp.float32)

    @pl.when(i == GRID - 1)
    def _finalize():
        loss_ref[...] = acc_s[...] * ((1.0 + BETA) / (B_TOTAL * E_DIM))
        e_mean = counts_s[...] * (1.0 / B_TOTAL)
        ent = jnp.sum(e_mean * jnp.log(e_mean + 1e-10),
                      axis=(0, 1), keepdims=True)
        perp_ref[...] = jnp.exp(-ent)


def kernel(z, codebook, interpret=False):
    z2 = z.reshape(B_TOTAL, E_DIM)
    zt = z2.T
    zq, idx3, loss, perp = pl.pallas_call(
        _vq_body,
        grid=(GRID,),
        in_specs=[
            pl.BlockSpec((E_DIM, BLK), lambda i: (0, i)),
            pl.BlockSpec((BLK, E_DIM), lambda i: (i, 0)),
            pl.BlockSpec((N_E, E_DIM), lambda i: (0, 0)),
        ],
        out_specs=[
            pl.BlockSpec((BLK, E_DIM), lambda i: (i, 0)),
            pl.BlockSpec((1, 1, BLK), lambda i: (i, 0, 0)),
            pl.BlockSpec((1, 1), lambda i: (0, 0)),
            pl.BlockSpec((1, 1), lambda i: (0, 0)),
        ],
        out_shape=[
            jax.ShapeDtypeStruct((B_TOTAL, E_DIM), jnp.float32),
            jax.ShapeDtypeStruct((GRID, 1, BLK), jnp.int32),
            jax.ShapeDtypeStruct((1, 1), jnp.float32),
            jax.ShapeDtypeStruct((1, 1), jnp.float32),
        ],
        scratch_shapes=[
            pltpu.VMEM((N_E, 1), jnp.float32),
            pltpu.VMEM((1, 1), jnp.float32),
        ],
        interpret=interpret,
    )(zt, z2, codebook)
    idx = idx3.reshape(B_TOTAL)[:, None]
    return (zq, loss.reshape(()), idx, perp.reshape(()))


# speedup vs baseline: 2.4986x; 1.4867x over previous
import jax
import jax.numpy as jnp
from jax.experimental import pallas as pl
from jax.experimental.pallas import tpu as pltpu

N_E = 1024
E_DIM = 64
BETA = 0.25
B_TOTAL = 36864
BLK = 4096
GRID = B_TOTAL // BLK


def _vq_body(zt_ref, c_ref, zqt_ref, idx_ref, loss_ref, perp_ref,
             counts_s, acc_s):
    i = pl.program_id(0)

    @pl.when(i == 0)
    def _init():
        counts_s[...] = jnp.zeros_like(counts_s)
        acc_s[...] = jnp.zeros_like(acc_s)

    c = c_ref[...]
    c_bf = c.astype(jnp.bfloat16)
    cnorm_c = jnp.sum(c * c, axis=1, keepdims=True)
    iota_c = jax.lax.broadcasted_iota(jnp.int32, (N_E, 1), 0).astype(
        jnp.float32)

    zt = zt_ref[...]

    znorm_t = jnp.sum(zt * zt, axis=0, keepdims=True)

    sneg2_t = jax.lax.dot_general(-2.0 * c, zt,
                                  (((1,), (0,)), ((), ())),
                                  preferred_element_type=jnp.float32)
    d_t = (znorm_t + cnorm_c) + sneg2_t

    dmin_t = jnp.min(d_t, axis=0, keepdims=True)
    idxrow_f = jnp.min(jnp.where(d_t == dmin_t, iota_c, float(N_E)),
                       axis=0, keepdims=True)

    one_hot_t = (iota_c == idxrow_f).astype(jnp.bfloat16)
    zq_t = jax.lax.dot_general(c_bf, one_hot_t,
                               (((0,), (0,)), ((), ())),
                               preferred_element_type=jnp.float32)

    zqt_ref[...] = zt + (zq_t - zt)
    idx_ref[...] = idxrow_f.astype(jnp.int32).reshape(1, 1, BLK)

    diff = zq_t - zt
    acc_s[...] += jnp.sum(diff * diff, axis=(0, 1), keepdims=True)
    counts_s[...] += jnp.sum(one_hot_t, axis=1, keepdims=True,
                             dtype=jnp.float32)

    @pl.when(i == GRID - 1)
    def _finalize():
        loss_ref[...] = acc_s[...] * ((1.0 + BETA) / (B_TOTAL * E_DIM))
        e_mean = counts_s[...] * (1.0 / B_TOTAL)
        ent = jnp.sum(e_mean * jnp.log(e_mean + 1e-10),
                      axis=(0, 1), keepdims=True)
        perp_ref[...] = jnp.exp(-ent)


def kernel(z, codebook, interpret=False):
    z2 = z.reshape(B_TOTAL, E_DIM)
    zt = z2.T
    zqt, idx3, loss, perp = pl.pallas_call(
        _vq_body,
        grid=(GRID,),
        in_specs=[
            pl.BlockSpec((E_DIM, BLK), lambda i: (0, i)),
            pl.BlockSpec((N_E, E_DIM), lambda i: (0, 0)),
        ],
        out_specs=[
            pl.BlockSpec((E_DIM, BLK), lambda i: (0, i)),
            pl.BlockSpec((1, 1, BLK), lambda i: (i, 0, 0)),
            pl.BlockSpec((1, 1), lambda i: (0, 0)),
            pl.BlockSpec((1, 1), lambda i: (0, 0)),
        ],
        out_shape=[
            jax.ShapeDtypeStruct((E_DIM, B_TOTAL), jnp.float32),
            jax.ShapeDtypeStruct((GRID, 1, BLK), jnp.int32),
            jax.ShapeDtypeStruct((1, 1), jnp.float32),
            jax.ShapeDtypeStruct((1, 1), jnp.float32),
        ],
        scratch_shapes=[
            pltpu.VMEM((N_E, 1), jnp.float32),
            pltpu.VMEM((1, 1), jnp.float32),
        ],
        interpret=interpret,
    )(zt, codebook)
    idx = idx3.reshape(B_TOTAL)[:, None]
    return (zqt.T, loss.reshape(()), idx, perp.reshape(()))


# speedup vs baseline: 2.5041x; 1.0022x over previous
import jax
import jax.numpy as jnp
from jax.experimental import pallas as pl
from jax.experimental.pallas import tpu as pltpu

N_E = 1024
E_DIM = 64
BETA = 0.25
B_TOTAL = 36864
BLK = 6144
GRID = B_TOTAL // BLK


def _vq_body(zt_ref, c_ref, zqt_ref, idx_ref, loss_ref, perp_ref,
             counts_s, acc_s):
    i = pl.program_id(0)

    @pl.when(i == 0)
    def _init():
        counts_s[...] = jnp.zeros_like(counts_s)
        acc_s[...] = jnp.zeros_like(acc_s)

    c = c_ref[...]
    c_bf = c.astype(jnp.bfloat16)
    cnorm_c = jnp.sum(c * c, axis=1, keepdims=True)
    iota_c = jax.lax.broadcasted_iota(jnp.int32, (N_E, 1), 0).astype(
        jnp.float32)

    zt = zt_ref[...]

    znorm_t = jnp.sum(zt * zt, axis=0, keepdims=True)

    sneg2_t = jax.lax.dot_general(-2.0 * c, zt,
                                  (((1,), (0,)), ((), ())),
                                  preferred_element_type=jnp.float32)
    d_t = (znorm_t + cnorm_c) + sneg2_t

    dmin_t = jnp.min(d_t, axis=0, keepdims=True)
    idxrow_f = jnp.min(jnp.where(d_t == dmin_t, iota_c, float(N_E)),
                       axis=0, keepdims=True)

    one_hot_t = (iota_c == idxrow_f).astype(jnp.bfloat16)
    zq_t = jax.lax.dot_general(c_bf, one_hot_t,
                               (((0,), (0,)), ((), ())),
                               preferred_element_type=jnp.float32)

    zqt_ref[...] = zt + (zq_t - zt)
    idx_ref[...] = idxrow_f.astype(jnp.int32).reshape(1, 1, BLK)

    diff = zq_t - zt
    acc_s[...] += jnp.sum(diff * diff, axis=(0, 1), keepdims=True)
    counts_s[...] += jnp.sum(one_hot_t, axis=1, keepdims=True,
                             dtype=jnp.float32)

    @pl.when(i == GRID - 1)
    def _finalize():
        loss_ref[...] = acc_s[...] * ((1.0 + BETA) / (B_TOTAL * E_DIM))
        e_mean = counts_s[...] * (1.0 / B_TOTAL)
        ent = jnp.sum(e_mean * jnp.log(e_mean + 1e-10),
                      axis=(0, 1), keepdims=True)
        perp_ref[...] = jnp.exp(-ent)


def kernel(z, codebook, interpret=False):
    z2 = z.reshape(B_TOTAL, E_DIM)
    zt = z2.T
    zqt, idx3, loss, perp = pl.pallas_call(
        _vq_body,
        grid=(GRID,),
        in_specs=[
            pl.BlockSpec((E_DIM, BLK), lambda i: (0, i)),
            pl.BlockSpec((N_E, E_DIM), lambda i: (0, 0)),
        ],
        out_specs=[
            pl.BlockSpec((E_DIM, BLK), lambda i: (0, i)),
            pl.BlockSpec((1, 1, BLK), lambda i: (i, 0, 0)),
            pl.BlockSpec((1, 1), lambda i: (0, 0)),
            pl.BlockSpec((1, 1), lambda i: (0, 0)),
        ],
        out_shape=[
            jax.ShapeDtypeStruct((E_DIM, B_TOTAL), jnp.float32),
            jax.ShapeDtypeStruct((GRID, 1, BLK), jnp.int32),
            jax.ShapeDtypeStruct((1, 1), jnp.float32),
            jax.ShapeDtypeStruct((1, 1), jnp.float32),
        ],
        scratch_shapes=[
            pltpu.VMEM((N_E, 1), jnp.float32),
            pltpu.VMEM((1, 1), jnp.float32),
        ],
        interpret=interpret,
    )(zt, codebook)
    idx = idx3.reshape(B_TOTAL)[:, None]
    return (zqt.T, loss.reshape(()), idx, perp.reshape(()))
